# baseline (speedup 1.0000x reference)
import functools

import jax
import jax.numpy as jnp
from jax import lax
from jax.experimental import pallas as pl
from jax.experimental.pallas import tpu as pltpu
from jax.experimental.pallas import tpu_sc as plsc


def _knn_body(fine_ref, coarse_t_ref, gidx_ref, wn_ref, *, bm, kk):
    x = fine_ref[0]
    y = coarse_t_ref[0]

    x0 = x[:, 0:1]
    x1 = x[:, 1:2]
    x2c = x[:, 2:3]
    y0 = y[0:1, :]
    y1 = y[1:2, :]
    y2c = y[2:3, :]

    xsq = x0 * x0 + x1 * x1 + x2c * x2c
    ysq = y0 * y0 + y1 * y1 + y2c * y2c
    xy = jnp.dot(x.astype(jnp.bfloat16), y.astype(jnp.bfloat16),
                 preferred_element_type=jnp.float32)
    r = xy - 0.5 * ysq

    ninf = jnp.float32(-jnp.inf)
    v1 = jnp.max(r, axis=1, keepdims=True)
    lt1 = r < v1
    v2 = jnp.max(jnp.where(lt1, r, ninf), axis=1, keepdims=True)
    lt2 = r < v2
    v3 = jnp.max(jnp.where(lt2, r, ninf), axis=1, keepdims=True)
    lt3 = r < v3

    iota = lax.broadcasted_iota(jnp.int32, (bm, kk), 1)
    big = jnp.int32(kk)
    i1 = jnp.min(jnp.where(lt1, big, iota), axis=1, keepdims=True)
    i2 = jnp.min(jnp.where(lt1 & (~lt2), iota, big), axis=1, keepdims=True)
    i3 = jnp.min(jnp.where(lt2 & (~lt3), iota, big), axis=1, keepdims=True)

    def w_of(v):
        return lax.rsqrt(jnp.maximum(xsq - 2.0 * v, 1e-12))

    w1, w2, w3 = w_of(v1), w_of(v2), w_of(v3)
    rws = 1.0 / (w1 + w2 + w3)
    base = pl.program_id(0) * kk
    gidx_ref[0] = jnp.concatenate([i1, i2, i3], axis=1) + base
    wn_ref[0] = jnp.concatenate([w1, w2, w3], axis=1) * rws


def _sc_gather(feats_hbm, gidx_hbm, w_hbm, out_hbm, idx_v, rows_v, w_v,
               out_v, sem):
    NC = 2
    wid = lax.axis_index("s") * NC + lax.axis_index("c")
    P = 2048
    CH = 16

    def body(i, carry):
        base_pt = wid * P + i * CH
        base_row = 3 * base_pt
        pltpu.sync_copy(gidx_hbm.at[pl.ds(base_row, 3 * CH)], idx_v)
        pltpu.sync_copy(w_hbm.at[pl.ds(base_row, 3 * CH)], w_v)
        pltpu.async_copy(feats_hbm.at[idx_v], rows_v, sem).wait()
        for p in range(CH):
            wv = [w_v[3 * p + j, :] for j in range(3)]
            for t in range(16):
                sl = pl.ds(16 * t, 16)
                acc = (rows_v[3 * p, sl] * wv[0]
                       + rows_v[3 * p + 1, sl] * wv[1]
                       + rows_v[3 * p + 2, sl] * wv[2])
                out_v[p, sl] = acc
        pltpu.sync_copy(out_v, out_hbm.at[pl.ds(base_pt, CH)])
        return carry

    lax.fori_loop(0, P // CH, body, 0)


def _mlp_body(x_ref, w1t_ref, b1_ref, w2t_ref, b2_ref, out_ref):
    xm = x_ref[...]
    h = xm @ w1t_ref[...] + b1_ref[...]
    h = 0.5 * h * (1.0 + lax.erf(h * jnp.float32(0.7071067811865476)))
    out_ref[...] = h @ w2t_ref[...] + b2_ref[...]


def kernel(fine_coords, coarse_coords, coarse_feats, W1, b1, W2, b2):
    B, M, _ = fine_coords.shape
    _, K, C = coarse_feats.shape
    O = W1.shape[0]
    BM = M

    coarse_t = coarse_coords.transpose(0, 2, 1)

    gidx, wn = pl.pallas_call(
        functools.partial(_knn_body, bm=BM, kk=K),
        grid=(B,),
        in_specs=[
            pl.BlockSpec((1, BM, 3), lambda b: (b, 0, 0)),
            pl.BlockSpec((1, 3, K), lambda b: (b, 0, 0)),
        ],
        out_specs=[
            pl.BlockSpec((1, BM, 3), lambda b: (b, 0, 0)),
            pl.BlockSpec((1, BM, 3), lambda b: (b, 0, 0)),
        ],
        out_shape=[
            jax.ShapeDtypeStruct((B, M, 3), jnp.int32),
            jax.ShapeDtypeStruct((B, M, 3), jnp.float32),
        ],
    )(fine_coords, coarse_t)

    feats_flat = coarse_feats.reshape(B * K, C)
    gidx_flat = gidx.reshape(B * M * 3)
    w_flat = jnp.broadcast_to(wn.reshape(B * M * 3, 1), (B * M * 3, 16))
    w_flat = jnp.asarray(w_flat)

    mesh = plsc.VectorSubcoreMesh(core_axis_name="c", subcore_axis_name="s")
    interp = pl.kernel(
        _sc_gather,
        mesh=mesh,
        out_type=jax.ShapeDtypeStruct((B * M, C), jnp.float32),
        scratch_types=[
            pltpu.VMEM((48,), jnp.int32),
            pltpu.VMEM((48, 256), jnp.float32),
            pltpu.VMEM((48, 16), jnp.float32),
            pltpu.VMEM((16, 256), jnp.float32),
            pltpu.SemaphoreType.DMA,
        ],
    )(feats_flat, gidx_flat, w_flat)

    RB = 4096
    out = pl.pallas_call(
        _mlp_body,
        grid=(B * M // RB,),
        in_specs=[
            pl.BlockSpec((RB, C), lambda i: (i, 0)),
            pl.BlockSpec((C, O), lambda i: (0, 0)),
            pl.BlockSpec((1, O), lambda i: (0, 0)),
            pl.BlockSpec((O, O), lambda i: (0, 0)),
            pl.BlockSpec((1, O), lambda i: (0, 0)),
        ],
        out_specs=pl.BlockSpec((RB, O), lambda i: (i, 0)),
        out_shape=jax.ShapeDtypeStruct((B * M, O), jnp.float32),
    )(interp, W1.T, b1.reshape(1, O), W2.T, b2.reshape(1, O))
    return out.reshape(B, M, O)
